# baseline (speedup 1.0000x reference)
import functools

import jax
import jax.numpy as jnp
from jax import lax
from jax.experimental import pallas as pl
from jax.experimental.pallas import tpu as pltpu
from jax.experimental.pallas import tpu_sc as plsc

NC, NS = 2, 16
NW = NC * NS
BB = 128


def _make_kernel(batch, seq, D):
    n_tc = batch // BB
    n_tr = D // 8
    b_per_w = BB * seq
    mesh = plsc.VectorSubcoreMesh(
        core_axis_name="c", subcore_axis_name="s", num_cores=NC, num_subcores=NS
    )

    @functools.partial(
        pl.kernel,
        out_type=jax.ShapeDtypeStruct((seq, n_tr, n_tc, 8 * BB), jnp.float32),
        mesh=mesh,
        scratch_types=[
            pltpu.VMEM((b_per_w,), jnp.int32),
            pltpu.VMEM((seq, BB), jnp.int32),
            pltpu.VMEM((2, BB, D), jnp.float32),
            pltpu.VMEM((2, n_tr * 8 * BB), jnp.float32),
            pltpu.SemaphoreType.DMA,
            pltpu.SemaphoreType.DMA,
        ],
        compiler_params=pltpu.CompilerParams(
            use_tc_tiling_on_sc=False, needs_layout_passes=False
        ),
    )
    def emb_kernel(idx_hbm, table_hbm, out_hbm, idx_v, idsT, gbuf, tbuf, gsem, ssem):
        wid = lax.axis_index("s") * NC + lax.axis_index("c")
        base = wid * b_per_w
        pltpu.sync_copy(idx_hbm.at[pl.ds(base, b_per_w)], idx_v)

        lane = lax.iota(jnp.int32, 16)
        biota = lane * seq
        pvecs = [(lane + 16 * p) * BB for p in range(D // 16)]

        @plsc.parallel_loop(0, seq, step=1, unroll=2)
        def _ids_body(s):
            for k in range(BB // 16):
                v = plsc.load_gather(idx_v, [biota + (16 * k * seq + s)])
                idsT[s, pl.ds(16 * k, 16)] = v

        def start_gather(s):
            pltpu.async_copy(table_hbm.at[idsT.at[s]], gbuf.at[s % 2], gsem)

        def wait_gather_one():
            pltpu.make_async_copy(
                table_hbm.at[pl.ds(0, BB)], gbuf.at[0], gsem
            ).wait()

        def wait_store_unit():
            pltpu.make_async_copy(
                tbuf.at[0, pl.ds(0, 8 * BB)], out_hbm.at[0, 0, 0], ssem
            ).wait()

        def transpose_block(h):
            @plsc.parallel_loop(0, BB, step=1, unroll=4)
            def _t_body(t):
                for p in range(D // 16):
                    v = gbuf[h, t, pl.ds(16 * p, 16)]
                    plsc.store_scatter(tbuf.at[h], [pvecs[p] + t], v)

        def start_store(s, h):
            for tr in range(n_tr):
                pltpu.async_copy(
                    tbuf.at[h, pl.ds(tr * 8 * BB, 8 * BB)],
                    out_hbm.at[s, tr, wid],
                    ssem,
                )

        start_gather(0)

        def step(s, h):
            wait_gather_one()

            @pl.when(s < seq - 1)
            def _():
                start_gather(s + 1)

            transpose_block(h)

            @pl.when(s >= 2)
            def _():
                for _u in range(n_tr):
                    wait_store_unit()

            start_store(s, h)

        def main_body(i, carry):
            step(2 * i, 0)
            step(2 * i + 1, 1)
            return carry

        lax.fori_loop(0, seq // 2, main_body, 0)

        for _u in range(2 * n_tr):
            wait_store_unit()

    return emb_kernel


@jax.jit
def kernel(input_ids, weight):
    batch, seq = input_ids.shape
    V, D = weight.shape
    B = batch * seq
    idx = input_ids.reshape(B).astype(jnp.int32)
    one = lax.optimization_barrier(jnp.float32(1.0))
    out5 = _make_kernel(batch, seq, D)(idx, weight * one)
    out5 = out5.reshape(seq, D // 8, batch // BB, 8, BB)
    return out5.transpose(2, 4, 0, 1, 3).reshape(batch, seq, D)


# speedup vs baseline: 1.2429x; 1.2429x over previous
import functools

import jax
import jax.numpy as jnp
from jax import lax
from jax.experimental import pallas as pl
from jax.experimental.pallas import tpu as pltpu
from jax.experimental.pallas import tpu_sc as plsc

NC, NS = 2, 16
NW = NC * NS
BB = 128


def _make_kernel(batch, seq, D):
    n_tc = batch // BB
    n_tr = D // 8
    b_per_w = BB * seq
    mesh = plsc.VectorSubcoreMesh(
        core_axis_name="c", subcore_axis_name="s", num_cores=NC, num_subcores=NS
    )

    @functools.partial(
        pl.kernel,
        out_type=jax.ShapeDtypeStruct((seq, n_tr, n_tc, 8 * BB), jnp.float32),
        mesh=mesh,
        scratch_types=[
            pltpu.VMEM((b_per_w,), jnp.int32),
            pltpu.VMEM((seq, BB), jnp.int32),
            pltpu.VMEM((2, BB, D), jnp.float32),
            pltpu.VMEM((2, n_tr * 8 * BB), jnp.float32),
            pltpu.SemaphoreType.DMA,
            pltpu.SemaphoreType.DMA,
        ],
        compiler_params=pltpu.CompilerParams(
            use_tc_tiling_on_sc=False, needs_layout_passes=False
        ),
    )
    def emb_kernel(idx_hbm, table_hbm, out_hbm, idx_v, idsT, gbuf, tbuf, gsem, ssem):
        wid = lax.axis_index("s") * NC + lax.axis_index("c")
        base = wid * b_per_w
        pltpu.sync_copy(idx_hbm.at[pl.ds(base, b_per_w)], idx_v)

        lane = lax.iota(jnp.int32, 16)
        biota = lane * seq
        pvecs = [(lane + 16 * p) * BB for p in range(D // 16)]

        @plsc.parallel_loop(0, seq, step=1, unroll=2)
        def _ids_body(s):
            for k in range(BB // 16):
                v = plsc.load_gather(idx_v, [biota + (16 * k * seq + s)])
                idsT[s, pl.ds(16 * k, 16)] = v

        def start_gather(s):
            pltpu.async_copy(table_hbm.at[idsT.at[s]], gbuf.at[s % 2], gsem)

        def wait_gather_one():
            pltpu.make_async_copy(
                table_hbm.at[pl.ds(0, BB)], gbuf.at[0], gsem
            ).wait()

        def wait_store_unit():
            pltpu.make_async_copy(
                tbuf.at[0, pl.ds(0, 8 * BB)], out_hbm.at[0, 0, 0], ssem
            ).wait()

        def transpose_block(h):
            @plsc.parallel_loop(0, BB, step=1, unroll=8)
            def _t_body(t):
                for p in range(D // 16):
                    v = gbuf[h, t, pl.ds(16 * p, 16)]
                    plsc.store_scatter(tbuf.at[h], [pvecs[p] + t], v)

        def start_store(s, h):
            for tr in range(n_tr):
                pltpu.async_copy(
                    tbuf.at[h, pl.ds(tr * 8 * BB, 8 * BB)],
                    out_hbm.at[s, tr, wid],
                    ssem,
                )

        start_gather(0)

        def step(s, h):
            wait_gather_one()

            @pl.when(s < seq - 1)
            def _():
                start_gather(s + 1)

            transpose_block(h)

            @pl.when(s >= 2)
            def _():
                for _u in range(n_tr):
                    wait_store_unit()

            start_store(s, h)

        def main_body(i, carry):
            step(2 * i, 0)
            step(2 * i + 1, 1)
            return carry

        lax.fori_loop(0, seq // 2, main_body, 0)

        for _u in range(2 * n_tr):
            wait_store_unit()

    return emb_kernel


@jax.jit
def kernel(input_ids, weight):
    batch, seq = input_ids.shape
    V, D = weight.shape
    B = batch * seq
    idx = input_ids.reshape(B).astype(jnp.int32)
    out5 = _make_kernel(batch, seq, D)(idx, weight)
    out5 = out5.reshape(seq, D // 8, batch // BB, 8, BB)
    return out5.transpose(2, 4, 0, 1, 3).reshape(batch, seq, D)


# speedup vs baseline: 1.9280x; 1.5512x over previous
import functools

import jax
import jax.numpy as jnp
from jax import lax
from jax.experimental import pallas as pl
from jax.experimental.pallas import tpu as pltpu
from jax.experimental.pallas import tpu_sc as plsc

NC, NS = 2, 16
NW = NC * NS
BB = 128


def _make_kernel(batch, seq, D):
    n_tc = batch // BB
    n_tr = D // 8
    b_per_w = BB * seq
    mesh = plsc.VectorSubcoreMesh(
        core_axis_name="c", subcore_axis_name="s", num_cores=NC, num_subcores=NS
    )

    @functools.partial(
        pl.kernel,
        out_type=jax.ShapeDtypeStruct((seq, n_tr, n_tc, 8, BB), jnp.float32),
        mesh=mesh,
        scratch_types=[
            pltpu.VMEM((b_per_w,), jnp.int32),
            pltpu.VMEM((seq, BB), jnp.int32),
            pltpu.VMEM((2, BB, D), jnp.float32),
            pltpu.VMEM((2, n_tr * 8, BB + 1), jnp.float32),
            pltpu.SemaphoreType.DMA,
            pltpu.SemaphoreType.DMA,
        ],
        compiler_params=pltpu.CompilerParams(
            use_tc_tiling_on_sc=False, needs_layout_passes=False
        ),
    )
    def emb_kernel(idx_hbm, table_hbm, out_hbm, idx_v, idsT, gbuf, tbuf, gsem, ssem):
        wid = lax.axis_index("s") * NC + lax.axis_index("c")
        base = wid * b_per_w
        pltpu.sync_copy(idx_hbm.at[pl.ds(base, b_per_w)], idx_v)

        lane = lax.iota(jnp.int32, 16)
        biota = lane * seq
        pvecs = [lane + 16 * p for p in range(D // 16)]

        @plsc.parallel_loop(0, seq, step=1, unroll=2)
        def _ids_body(s):
            for k in range(BB // 16):
                v = plsc.load_gather(idx_v, [biota + (16 * k * seq + s)])
                idsT[s, pl.ds(16 * k, 16)] = v

        def start_gather(s):
            pltpu.async_copy(table_hbm.at[idsT.at[s]], gbuf.at[s % 2], gsem)

        def wait_gather_one():
            pltpu.make_async_copy(
                table_hbm.at[pl.ds(0, BB)], gbuf.at[0], gsem
            ).wait()

        def wait_store_unit():
            pltpu.make_async_copy(
                tbuf.at[0, pl.ds(0, 8), pl.ds(0, BB)], out_hbm.at[0, 0, 0], ssem
            ).wait()

        def transpose_block(h):
            @plsc.parallel_loop(0, BB, step=1, unroll=8)
            def _t_body(t):
                tvec = jnp.full((16,), 0, jnp.int32) + t
                for p in range(D // 16):
                    v = gbuf[h, t, pl.ds(16 * p, 16)]
                    plsc.store_scatter(tbuf.at[h], [pvecs[p], tvec], v)

        def start_store(s, h):
            for tr in range(n_tr):
                pltpu.async_copy(
                    tbuf.at[h, pl.ds(tr * 8, 8), pl.ds(0, BB)],
                    out_hbm.at[s, tr, wid],
                    ssem,
                )

        start_gather(0)

        def step(s, h):
            wait_gather_one()

            @pl.when(s < seq - 1)
            def _():
                start_gather(s + 1)

            transpose_block(h)

            @pl.when(s >= 2)
            def _():
                for _u in range(n_tr):
                    wait_store_unit()

            start_store(s, h)

        def main_body(i, carry):
            step(2 * i, 0)
            step(2 * i + 1, 1)
            return carry

        lax.fori_loop(0, seq // 2, main_body, 0)

        for _u in range(2 * n_tr):
            wait_store_unit()

    return emb_kernel


@jax.jit
def kernel(input_ids, weight):
    batch, seq = input_ids.shape
    V, D = weight.shape
    B = batch * seq
    idx = input_ids.reshape(B).astype(jnp.int32)
    out5 = _make_kernel(batch, seq, D)(idx, weight)
    return out5.transpose(2, 4, 0, 1, 3).reshape(batch, seq, D)


# speedup vs baseline: 1.9283x; 1.0001x over previous
import functools

import jax
import jax.numpy as jnp
from jax import lax
from jax.experimental import pallas as pl
from jax.experimental.pallas import tpu as pltpu
from jax.experimental.pallas import tpu_sc as plsc

NC, NS = 2, 16
NW = NC * NS
BB = 128


def _make_kernel(batch, seq, D):
    n_tc = batch // BB
    n_tr = D // 8
    b_per_w = BB * seq
    mesh = plsc.VectorSubcoreMesh(
        core_axis_name="c", subcore_axis_name="s", num_cores=NC, num_subcores=NS
    )

    @functools.partial(
        pl.kernel,
        out_type=jax.ShapeDtypeStruct((seq, n_tr, n_tc, 8, BB), jnp.float32),
        mesh=mesh,
        scratch_types=[
            pltpu.VMEM((b_per_w,), jnp.int32),
            pltpu.VMEM((seq, BB), jnp.int32),
            pltpu.VMEM((2, BB, D), jnp.float32),
            pltpu.VMEM((2, n_tr, 8, BB + 1), jnp.float32),
            pltpu.SemaphoreType.DMA,
            pltpu.SemaphoreType.DMA,
        ],
        compiler_params=pltpu.CompilerParams(
            use_tc_tiling_on_sc=False, needs_layout_passes=False
        ),
    )
    def emb_kernel(idx_hbm, table_hbm, out_hbm, idx_v, idsT, gbuf, tbuf, gsem, ssem):
        wid = lax.axis_index("s") * NC + lax.axis_index("c")
        base = wid * b_per_w
        pltpu.sync_copy(idx_hbm.at[pl.ds(base, b_per_w)], idx_v)

        lane = lax.iota(jnp.int32, 16)
        biota = lane * seq
        pvecs = [lane + 16 * p for p in range(D // 16)]
        trvecs = [lax.shift_right_logical(pvecs[p], 3) for p in range(D // 16)]
        jrvecs = [lax.bitwise_and(pvecs[p], 7) for p in range(D // 16)]

        @plsc.parallel_loop(0, seq, step=1, unroll=2)
        def _ids_body(s):
            for k in range(BB // 16):
                v = plsc.load_gather(idx_v, [biota + (16 * k * seq + s)])
                idsT[s, pl.ds(16 * k, 16)] = v

        def start_gather(s):
            pltpu.async_copy(table_hbm.at[idsT.at[s]], gbuf.at[s % 2], gsem)

        def wait_gather_one():
            pltpu.make_async_copy(
                table_hbm.at[pl.ds(0, BB)], gbuf.at[0], gsem
            ).wait()

        def wait_store_unit():
            pltpu.make_async_copy(
                tbuf.at[0, :, :, pl.ds(0, BB)], out_hbm.at[0, :, 0], ssem
            ).wait()

        def transpose_block(h):
            @plsc.parallel_loop(0, BB, step=1, unroll=8)
            def _t_body(t):
                tvec = jnp.full((16,), 0, jnp.int32) + t
                for p in range(D // 16):
                    v = gbuf[h, t, pl.ds(16 * p, 16)]
                    plsc.store_scatter(tbuf.at[h], [trvecs[p], jrvecs[p], tvec], v)

        def start_store(s, h):
            pltpu.async_copy(
                tbuf.at[h, :, :, pl.ds(0, BB)],
                out_hbm.at[s, :, wid],
                ssem,
            )

        start_gather(0)

        def step(s, h):
            wait_gather_one()

            @pl.when(s < seq - 1)
            def _():
                start_gather(s + 1)

            transpose_block(h)

            @pl.when(s >= 2)
            def _():
                wait_store_unit()

            start_store(s, h)

        def main_body(i, carry):
            step(2 * i, 0)
            step(2 * i + 1, 1)
            return carry

        lax.fori_loop(0, seq // 2, main_body, 0)

        for _u in range(2):
            wait_store_unit()

    return emb_kernel


@jax.jit
def kernel(input_ids, weight):
    batch, seq = input_ids.shape
    V, D = weight.shape
    B = batch * seq
    idx = input_ids.reshape(B).astype(jnp.int32)
    out5 = _make_kernel(batch, seq, D)(idx, weight)
    return out5.transpose(2, 4, 0, 1, 3).reshape(batch, seq, D)


# speedup vs baseline: 2.1410x; 1.1103x over previous
import functools

import jax
import jax.numpy as jnp
from jax import lax
from jax.experimental import pallas as pl
from jax.experimental.pallas import tpu as pltpu
from jax.experimental.pallas import tpu_sc as plsc

NC, NS = 2, 16
NW = NC * NS
BB = 128


def _make_kernel(batch, seq, D):
    n_tc = batch // BB
    n_tr = D // 8
    b_per_w = BB * seq
    mesh = plsc.VectorSubcoreMesh(
        core_axis_name="c", subcore_axis_name="s", num_cores=NC, num_subcores=NS
    )

    @functools.partial(
        pl.kernel,
        out_type=jax.ShapeDtypeStruct((seq, n_tr, n_tc, 8, BB), jnp.float32),
        mesh=mesh,
        scratch_types=[
            pltpu.VMEM((b_per_w,), jnp.int32),
            pltpu.VMEM((seq, BB), jnp.int32),
            pltpu.VMEM((4, BB, D), jnp.float32),
            pltpu.VMEM((2, n_tr, 8, BB + 1), jnp.float32),
            pltpu.SemaphoreType.DMA,
            pltpu.SemaphoreType.DMA,
        ],
        compiler_params=pltpu.CompilerParams(
            use_tc_tiling_on_sc=False, needs_layout_passes=False
        ),
    )
    def emb_kernel(idx_hbm, table_hbm, out_hbm, idx_v, idsT, gbuf, tbuf, gsem, ssem):
        wid = lax.axis_index("s") * NC + lax.axis_index("c")
        base = wid * b_per_w
        pltpu.sync_copy(idx_hbm.at[pl.ds(base, b_per_w)], idx_v)

        lane = lax.iota(jnp.int32, 16)
        biota = lane * seq
        pvecs = [lane + 16 * p for p in range(D // 16)]
        trvecs = [lax.shift_right_logical(pvecs[p], 3) for p in range(D // 16)]
        jrvecs = [lax.bitwise_and(pvecs[p], 7) for p in range(D // 16)]

        @plsc.parallel_loop(0, seq, step=1, unroll=2)
        def _ids_body(s):
            for k in range(BB // 16):
                v = plsc.load_gather(idx_v, [biota + (16 * k * seq + s)])
                idsT[s, pl.ds(16 * k, 16)] = v

        def start_gather(s):
            pltpu.async_copy(table_hbm.at[idsT.at[s]], gbuf.at[s % 4], gsem)

        def wait_gather_one():
            pltpu.make_async_copy(
                table_hbm.at[pl.ds(0, BB)], gbuf.at[0], gsem
            ).wait()

        def wait_store_unit():
            pltpu.make_async_copy(
                tbuf.at[0, :, :, pl.ds(0, BB)], out_hbm.at[0, :, 0], ssem
            ).wait()

        def transpose_block(s, h):
            g = s % 4
            @plsc.parallel_loop(0, BB, step=1, unroll=8)
            def _t_body(t):
                tvec = jnp.full((16,), 0, jnp.int32) + t
                for p in range(D // 16):
                    v = gbuf[g, t, pl.ds(16 * p, 16)]
                    plsc.store_scatter(tbuf.at[h], [trvecs[p], jrvecs[p], tvec], v)

        def start_store(s, h):
            pltpu.async_copy(
                tbuf.at[h, :, :, pl.ds(0, BB)],
                out_hbm.at[s, :, wid],
                ssem,
            )

        start_gather(0)
        start_gather(1)

        def step(s, h):
            wait_gather_one()

            @pl.when(s < seq - 2)
            def _():
                start_gather(s + 2)

            transpose_block(s, h)

            @pl.when(s >= 2)
            def _():
                wait_store_unit()

            start_store(s, h)

        def main_body(i, carry):
            step(2 * i, 0)
            step(2 * i + 1, 1)
            return carry

        lax.fori_loop(0, seq // 2, main_body, 0)

        for _u in range(2):
            wait_store_unit()

    return emb_kernel


@jax.jit
def kernel(input_ids, weight):
    batch, seq = input_ids.shape
    V, D = weight.shape
    B = batch * seq
    idx = input_ids.reshape(B).astype(jnp.int32)
    out5 = _make_kernel(batch, seq, D)(idx, weight)
    return out5.transpose(2, 4, 0, 1, 3).reshape(batch, seq, D)


# speedup vs baseline: 2.1636x; 1.0106x over previous
import functools

import jax
import jax.numpy as jnp
from jax import lax
from jax.experimental import pallas as pl
from jax.experimental.pallas import tpu as pltpu
from jax.experimental.pallas import tpu_sc as plsc

NC, NS = 2, 16
NW = NC * NS
BB = 128


def _make_kernel(batch, seq, D):
    n_tc = batch // BB
    n_tr = D // 8
    b_per_w = BB * seq
    mesh = plsc.VectorSubcoreMesh(
        core_axis_name="c", subcore_axis_name="s", num_cores=NC, num_subcores=NS
    )

    @functools.partial(
        pl.kernel,
        out_type=jax.ShapeDtypeStruct((seq, n_tr, n_tc, 8, BB), jnp.float32),
        mesh=mesh,
        scratch_types=[
            pltpu.VMEM((b_per_w,), jnp.int32),
            pltpu.VMEM((seq, BB), jnp.int32),
            pltpu.VMEM((4, BB, D), jnp.float32),
            pltpu.VMEM((2, n_tr, 8, BB + 1), jnp.float32),
            pltpu.SemaphoreType.DMA,
            pltpu.SemaphoreType.DMA,
        ],
        compiler_params=pltpu.CompilerParams(
            use_tc_tiling_on_sc=False, needs_layout_passes=False
        ),
    )
    def emb_kernel(idx_hbm, table_hbm, out_hbm, idx_v, idsT, gbuf, tbuf, gsem, ssem):
        wid = lax.axis_index("s") * NC + lax.axis_index("c")
        base = wid * b_per_w
        pltpu.sync_copy(idx_hbm.at[pl.ds(base, b_per_w)], idx_v)

        lane = lax.iota(jnp.int32, 16)
        biota = lane * seq
        pvecs = [lane + 16 * p for p in range(D // 16)]
        trvecs = [lax.shift_right_logical(pvecs[p], 3) for p in range(D // 16)]
        jrvecs = [lax.bitwise_and(pvecs[p], 7) for p in range(D // 16)]

        @plsc.parallel_loop(0, seq, step=1, unroll=2)
        def _ids_body(s):
            for k in range(BB // 16):
                v = plsc.load_gather(idx_v, [biota + (16 * k * seq + s)])
                idsT[s, pl.ds(16 * k, 16)] = v

        def start_gather(s):
            pltpu.async_copy(table_hbm.at[idsT.at[s]], gbuf.at[s % 4], gsem)

        def wait_gather_one():
            pltpu.make_async_copy(
                table_hbm.at[pl.ds(0, BB)], gbuf.at[0], gsem
            ).wait()

        def wait_store_unit():
            pltpu.make_async_copy(
                tbuf.at[0, :, :, pl.ds(0, BB)], out_hbm.at[0, :, 0], ssem
            ).wait()

        def transpose_block(s, h):
            g = s % 4
            @plsc.parallel_loop(0, BB, step=1, unroll=8)
            def _t_body(t):
                tvec = jnp.full((16,), 0, jnp.int32) + t
                for p in range(D // 16):
                    v = gbuf[g, t, pl.ds(16 * p, 16)]
                    plsc.store_scatter(tbuf.at[h], [trvecs[p], jrvecs[p], tvec], v)

        def start_store(s, h):
            pltpu.async_copy(
                tbuf.at[h, :, :, pl.ds(0, BB)],
                out_hbm.at[s, :, wid],
                ssem,
            )

        start_gather(0)
        start_gather(1)
        start_gather(2)

        def step(s, h):
            wait_gather_one()

            @pl.when(s < seq - 3)
            def _():
                start_gather(s + 3)

            transpose_block(s, h)

            @pl.when(s >= 2)
            def _():
                wait_store_unit()

            start_store(s, h)

        def main_body(i, carry):
            step(2 * i, 0)
            step(2 * i + 1, 1)
            return carry

        lax.fori_loop(0, seq // 2, main_body, 0)

        for _u in range(2):
            wait_store_unit()

    return emb_kernel


@jax.jit
def kernel(input_ids, weight):
    batch, seq = input_ids.shape
    V, D = weight.shape
    B = batch * seq
    idx = input_ids.reshape(B).astype(jnp.int32)
    out5 = _make_kernel(batch, seq, D)(idx, weight)
    return out5.transpose(2, 4, 0, 1, 3).reshape(batch, seq, D)
